# baseline (speedup 1.0000x reference)
import jax
import jax.numpy as jnp
from jax import lax
from jax.experimental import pallas as pl
from jax.experimental.pallas import tpu as pltpu
from jax.experimental.pallas import tpu_sc as plsc

ROWS, COLS = 16384, 200
TABLE = 1_000_000
N = ROWS * COLS
NC, NS, LANES = 2, 16, 16
NW = NC * NS
NPW = N // NW
CHUNK = 12800
NCHUNK = NPW // CHUNK
TCHUNK = 10_000
NTCHUNK = TABLE // TCHUNK
NJ = (NTCHUNK + NS - 1) // NS


def _sc_body(idx_hbm, lam_hbm, out_hbm,
             idx_a, idx_b, val_a, val_b, tab_sh,
             isem_a, isem_b, gsem_a, gsem_b, osem_a, osem_b):
    sid = lax.axis_index("s")
    cid = lax.axis_index("c")
    wid = sid * NC + cid

    idx_v = [idx_a, idx_b]
    val_v = [val_a, val_b]
    isem = [isem_a, isem_b]
    gsem = [gsem_a, gsem_b]
    osem = [osem_a, osem_b]
    base = wid * NPW

    def start_idx(c, b):
        off = base + c * CHUNK
        pltpu.async_copy(idx_hbm.at[pl.ds(off, CHUNK)], idx_v[b], isem[b])

    def wait_idx(b):
        pltpu.make_async_copy(
            idx_hbm.at[pl.ds(base, CHUNK)], idx_v[b], isem[b]
        ).wait()

    start_idx(0, 0)
    start_idx(1, 1)

    def tstage(b):
        return val_v[b].at[pl.ds(0, TCHUNK)]

    def tstart(j, b):
        o = (j * NS + sid) * TCHUNK

        @pl.when(j * NS + sid < NTCHUNK)
        def _():
            pltpu.async_copy(lam_hbm.at[pl.ds(o, TCHUNK)], tstage(b), gsem[b])

    def twait(j, b):
        @pl.when(j * NS + sid < NTCHUNK)
        def _():
            pltpu.make_async_copy(
                lam_hbm.at[pl.ds(0, TCHUNK)], tstage(b), gsem[b]
            ).wait()

    tstart(0, 0)
    for j in range(NJ):
        cur = j & 1
        nxt = 1 - cur
        if j + 1 < NJ:
            tstart(j + 1, nxt)
        twait(j, cur)

        @pl.when(j * NS + sid < NTCHUNK)
        def _():
            stage = tstage(cur)

            def fa(i, _):
                v = stage[pl.ds(i * LANES, LANES)]
                stage[pl.ds(i * LANES, LANES)] = jnp.where(
                    v >= 1.0, v, jnp.exp(v - 1.0)
                )
                return 0

            lax.fori_loop(0, TCHUNK // LANES, fa, 0, unroll=8)
            o = (j * NS + sid) * TCHUNK
            pltpu.sync_copy(stage, tab_sh.at[pl.ds(o, TCHUNK)])

    plsc.subcore_barrier()

    SP = 9600
    HB = CHUNK - SP

    def start_gather(b):
        pltpu.async_copy(
            tab_sh.at[idx_v[b].at[pl.ds(0, SP)]],
            val_v[b].at[pl.ds(0, SP)], gsem[b],
        )
        pltpu.async_copy(
            lam_hbm.at[idx_v[b].at[pl.ds(SP, HB)]],
            val_v[b].at[pl.ds(SP, HB)], isem[b],
        )

    def finish_gather(b):
        pltpu.make_async_copy(
            lam_hbm.at[idx_v[b].at[pl.ds(SP, HB)]],
            val_v[b].at[pl.ds(SP, HB)], isem[b],
        ).wait()

        def fb(i, _):
            v = val_v[b][pl.ds(SP + i * LANES, LANES)]
            val_v[b][pl.ds(SP + i * LANES, LANES)] = jnp.where(
                v >= 1.0, v, jnp.exp(v - 1.0)
            )
            return 0

        lax.fori_loop(0, HB // LANES, fb, 0, unroll=8)
        pltpu.make_async_copy(
            tab_sh.at[idx_v[b].at[pl.ds(0, SP)]],
            val_v[b].at[pl.ds(0, SP)], gsem[b],
        ).wait()

    def wait_out(b):
        pltpu.make_async_copy(
            val_v[b], out_hbm.at[pl.ds(base, CHUNK)], osem[b]
        ).wait()

    wait_idx(0)
    start_gather(0)

    for c in range(NCHUNK):
        cur = c & 1
        nxt = 1 - cur
        if c + 1 < NCHUNK:
            if c + 1 >= 2:
                start_idx(c + 1, nxt)
            wait_idx(nxt)
            if c + 1 >= 2:
                wait_out(nxt)
            start_gather(nxt)
        finish_gather(cur)
        off = base + c * CHUNK
        pltpu.async_copy(val_v[cur], out_hbm.at[pl.ds(off, CHUNK)], osem[cur])

    wait_out(0)
    wait_out(1)


def kernel(t_idx, lam):
    idx_flat = t_idx.reshape(N)
    mesh = plsc.VectorSubcoreMesh(core_axis_name="c", subcore_axis_name="s")
    out = pl.kernel(
        _sc_body,
        out_type=jax.ShapeDtypeStruct((N,), jnp.float32),
        mesh=mesh,
        scratch_types=[
            pltpu.VMEM((CHUNK,), jnp.int32),
            pltpu.VMEM((CHUNK,), jnp.int32),
            pltpu.VMEM((CHUNK,), jnp.float32),
            pltpu.VMEM((CHUNK,), jnp.float32),
            pltpu.MemorySpace.VMEM_SHARED((TABLE,), jnp.float32),
            pltpu.SemaphoreType.DMA,
            pltpu.SemaphoreType.DMA,
            pltpu.SemaphoreType.DMA,
            pltpu.SemaphoreType.DMA,
            pltpu.SemaphoreType.DMA,
            pltpu.SemaphoreType.DMA,
        ],
    )(idx_flat, lam)
    return out.reshape(ROWS, COLS)


# speedup vs baseline: 1.0522x; 1.0522x over previous
import jax
import jax.numpy as jnp
from jax import lax
from jax.experimental import pallas as pl
from jax.experimental.pallas import tpu as pltpu
from jax.experimental.pallas import tpu_sc as plsc

ROWS, COLS = 16384, 200
TABLE = 1_000_000
N = ROWS * COLS
NC, NS, LANES = 2, 16, 16
NW = NC * NS
NPW = N // NW
CHUNK = 12800
NCHUNK = NPW // CHUNK
TCHUNK = 10_000
NTCHUNK = TABLE // TCHUNK
NJ = (NTCHUNK + NS - 1) // NS


def _sc_body(idx_hbm, lam_hbm, out_hbm,
             idx_a, idx_b, val_a, val_b, tab_sh,
             isem_a, isem_b, gsem_a, gsem_b, osem_a, osem_b):
    sid = lax.axis_index("s")
    cid = lax.axis_index("c")
    wid = sid * NC + cid

    idx_v = [idx_a, idx_b]
    val_v = [val_a, val_b]
    isem = [isem_a, isem_b]
    gsem = [gsem_a, gsem_b]
    osem = [osem_a, osem_b]
    base = wid * NPW

    def start_idx(c, b):
        off = base + c * CHUNK
        pltpu.async_copy(idx_hbm.at[pl.ds(off, CHUNK)], idx_v[b], isem[b])

    def wait_idx(b):
        pltpu.make_async_copy(
            idx_hbm.at[pl.ds(base, CHUNK)], idx_v[b], isem[b]
        ).wait()

    start_idx(0, 0)
    start_idx(1, 1)

    def tstage(b):
        return val_v[b].at[pl.ds(0, TCHUNK)]

    def tstart(j, b):
        o = (j * NS + sid) * TCHUNK

        @pl.when(j * NS + sid < NTCHUNK)
        def _():
            pltpu.async_copy(lam_hbm.at[pl.ds(o, TCHUNK)], tstage(b), gsem[b])

    def twait(j, b):
        @pl.when(j * NS + sid < NTCHUNK)
        def _():
            pltpu.make_async_copy(
                lam_hbm.at[pl.ds(0, TCHUNK)], tstage(b), gsem[b]
            ).wait()

    tstart(0, 0)
    for j in range(NJ):
        cur = j & 1
        nxt = 1 - cur
        if j + 1 < NJ:
            tstart(j + 1, nxt)
        twait(j, cur)

        @pl.when(j * NS + sid < NTCHUNK)
        def _():
            stage = tstage(cur)

            def fa(i, _):
                v = stage[pl.ds(i * LANES, LANES)]
                stage[pl.ds(i * LANES, LANES)] = jnp.where(
                    v >= 1.0, v, jnp.exp(v - 1.0)
                )
                return 0

            lax.fori_loop(0, TCHUNK // LANES, fa, 0, unroll=8)
            o = (j * NS + sid) * TCHUNK
            pltpu.sync_copy(stage, tab_sh.at[pl.ds(o, TCHUNK)])

    plsc.subcore_barrier()

    def start_gather(b):
        pltpu.async_copy(tab_sh.at[idx_v[b]], val_v[b], gsem[b])

    def finish_gather(b):
        pltpu.make_async_copy(tab_sh.at[idx_v[b]], val_v[b], gsem[b]).wait()

    def wait_out(b):
        pltpu.make_async_copy(
            val_v[b], out_hbm.at[pl.ds(base, CHUNK)], osem[b]
        ).wait()

    wait_idx(0)
    start_gather(0)

    for c in range(NCHUNK):
        cur = c & 1
        nxt = 1 - cur
        if c + 1 < NCHUNK:
            if c + 1 >= 2:
                start_idx(c + 1, nxt)
            wait_idx(nxt)
            if c + 1 >= 2:
                wait_out(nxt)
            start_gather(nxt)
        finish_gather(cur)
        off = base + c * CHUNK
        pltpu.async_copy(val_v[cur], out_hbm.at[pl.ds(off, CHUNK)], osem[cur])

    wait_out(0)
    wait_out(1)


def kernel(t_idx, lam):
    idx_flat = t_idx.reshape(N)
    mesh = plsc.VectorSubcoreMesh(core_axis_name="c", subcore_axis_name="s")
    out = pl.kernel(
        _sc_body,
        out_type=jax.ShapeDtypeStruct((N,), jnp.float32),
        mesh=mesh,
        scratch_types=[
            pltpu.VMEM((CHUNK,), jnp.int32),
            pltpu.VMEM((CHUNK,), jnp.int32),
            pltpu.VMEM((CHUNK,), jnp.float32),
            pltpu.VMEM((CHUNK,), jnp.float32),
            pltpu.MemorySpace.VMEM_SHARED((TABLE,), jnp.float32),
            pltpu.SemaphoreType.DMA,
            pltpu.SemaphoreType.DMA,
            pltpu.SemaphoreType.DMA,
            pltpu.SemaphoreType.DMA,
            pltpu.SemaphoreType.DMA,
            pltpu.SemaphoreType.DMA,
        ],
    )(idx_flat, lam)
    return out.reshape(ROWS, COLS)
